# baseline (speedup 1.0000x reference)
import functools

import jax
import jax.numpy as jnp
from jax import lax
from jax.experimental import pallas as pl
from jax.experimental.pallas import tpu as pltpu
from jax.experimental.pallas import tpu_sc as plsc

N = 10000
E = 320000
D = 128
H = 32
G = 64
FH = 12
TV = 4

NP = 10240
NW = 32
CE = 128
EP = E + N
CPT = 82
EPAD = NW * CPT * CE
ROWS_PT = NP // 16

NB = 5
BN = N // NB


def _t1_body(x_ref, w_ref, as_ref, ad_ref, h_ref, als_ref, ald_ref,
             ma_ref, mb_ref):
    i = pl.program_id(0)
    h = x_ref[...] @ w_ref[...]
    h_ref[...] = h
    als = h @ as_ref[...]
    ald = h @ ad_ref[...]
    als_ref[...] = als
    ald_ref[...] = ald

    @pl.when(i == 0)
    def _():
        ma_ref[...] = jnp.full((1, 1), -jnp.inf, jnp.float32)
        mb_ref[...] = jnp.full((1, 1), -jnp.inf, jnp.float32)

    ma_ref[...] = jnp.maximum(ma_ref[...], jnp.max(als))
    mb_ref[...] = jnp.maximum(mb_ref[...], jnp.max(ald))


def _t1(x, W, a_s, a_d, din):
    return pl.pallas_call(
        _t1_body,
        grid=(NB,),
        in_specs=[
            pl.BlockSpec((BN, din), lambda i: (i, 0)),
            pl.BlockSpec((din, H), lambda i: (0, 0)),
            pl.BlockSpec((H, 1), lambda i: (0, 0)),
            pl.BlockSpec((H, 1), lambda i: (0, 0)),
        ],
        out_specs=[
            pl.BlockSpec((BN, H), lambda i: (i, 0)),
            pl.BlockSpec((BN, 1), lambda i: (i, 0)),
            pl.BlockSpec((BN, 1), lambda i: (i, 0)),
            pl.BlockSpec((1, 1), lambda i: (0, 0)),
            pl.BlockSpec((1, 1), lambda i: (0, 0)),
        ],
        out_shape=[
            jax.ShapeDtypeStruct((N, H), jnp.float32),
            jax.ShapeDtypeStruct((N, 1), jnp.float32),
            jax.ShapeDtypeStruct((N, 1), jnp.float32),
            jax.ShapeDtypeStruct((1, 1), jnp.float32),
            jax.ShapeDtypeStruct((1, 1), jnp.float32),
        ],
    )(x, W, a_s, a_d)


def _tmid_body(a0_ref, a1_ref, s0_ref, s1_ref, b_ref, w_ref, as_ref, ad_ref,
               h_ref, als_ref, ald_ref, ma_ref, mb_ref):
    i = pl.program_id(0)
    pre = (a0_ref[...] + a1_ref[...]) / (s0_ref[...] + s1_ref[...] + 1e-16)
    pre = pre + b_ref[...]
    act = 0.5 * pre * (1.0 + lax.erf(pre * (2.0 ** -0.5)))
    h = act @ w_ref[...]
    h_ref[...] = h
    als = h @ as_ref[...]
    ald = h @ ad_ref[...]
    als_ref[...] = als
    ald_ref[...] = ald

    @pl.when(i == 0)
    def _():
        ma_ref[...] = jnp.full((1, 1), -jnp.inf, jnp.float32)
        mb_ref[...] = jnp.full((1, 1), -jnp.inf, jnp.float32)

    ma_ref[...] = jnp.maximum(ma_ref[...], jnp.max(als))
    mb_ref[...] = jnp.maximum(mb_ref[...], jnp.max(ald))


def _tmid(a0, a1, s0, s1, b, W, a_s, a_d):
    return pl.pallas_call(
        _tmid_body,
        grid=(NB,),
        in_specs=[
            pl.BlockSpec((BN, H), lambda i: (i, 0)),
            pl.BlockSpec((BN, H), lambda i: (i, 0)),
            pl.BlockSpec((BN, 1), lambda i: (i, 0)),
            pl.BlockSpec((BN, 1), lambda i: (i, 0)),
            pl.BlockSpec((1, H), lambda i: (0, 0)),
            pl.BlockSpec((H, H), lambda i: (0, 0)),
            pl.BlockSpec((H, 1), lambda i: (0, 0)),
            pl.BlockSpec((H, 1), lambda i: (0, 0)),
        ],
        out_specs=[
            pl.BlockSpec((BN, H), lambda i: (i, 0)),
            pl.BlockSpec((BN, 1), lambda i: (i, 0)),
            pl.BlockSpec((BN, 1), lambda i: (i, 0)),
            pl.BlockSpec((1, 1), lambda i: (0, 0)),
            pl.BlockSpec((1, 1), lambda i: (0, 0)),
        ],
        out_shape=[
            jax.ShapeDtypeStruct((N, H), jnp.float32),
            jax.ShapeDtypeStruct((N, 1), jnp.float32),
            jax.ShapeDtypeStruct((N, 1), jnp.float32),
            jax.ShapeDtypeStruct((1, 1), jnp.float32),
            jax.ShapeDtypeStruct((1, 1), jnp.float32),
        ],
    )(a0, a1, s0, s1, b, W, a_s, a_d)


def _tf_body(a0_ref, a1_ref, s0_ref, s1_ref, bb_ref, b3_ref, wr_ref, br_ref,
             wm0_ref, bm0_ref, wm1_ref, bm1_ref, wl_ref, bl_ref,
             out_ref, sum_s, cnt_s, mx_s):
    i = pl.program_id(0)

    @pl.when(i == 0)
    def _():
        sum_s[...] = jnp.zeros((G, H), jnp.float32)
        cnt_s[...] = jnp.zeros((G, 1), jnp.float32)
        mx_s[...] = jnp.full((G, H), -jnp.inf, jnp.float32)

    h = (a0_ref[...] + a1_ref[...]) / (s0_ref[...] + s1_ref[...] + 1e-16)
    h = h + b3_ref[...]
    bb = bb_ref[...]
    oh = (bb == lax.broadcasted_iota(jnp.int32, (1, G), 1)).astype(jnp.float32)
    sum_s[...] += lax.dot_general(oh, h, (((0,), (0,)), ((), ())))
    cnt_s[...] += jnp.sum(oh, axis=0)[:, None]
    for g in range(G):
        mg = jnp.where(bb == g, h, -jnp.inf)
        mx_s[g:g + 1, :] = jnp.maximum(mx_s[g:g + 1, :],
                                       jnp.max(mg, axis=0, keepdims=True))

    @pl.when(i == NB - 1)
    def _():
        mean = sum_s[...] / jnp.maximum(cnt_s[...], 1.0)
        mxf = jnp.where(jnp.isfinite(mx_s[...]), mx_s[...], 0.0)
        z = jnp.concatenate([mean, mxf], axis=1)
        z = z @ wr_ref[...] + br_ref[...]
        z = jnp.maximum(z @ wm0_ref[...] + bm0_ref[...], 0.0)
        z = jnp.maximum(z @ wm1_ref[...] + bm1_ref[...], 0.0)
        out_ref[...] = z @ wl_ref[...] + bl_ref[...]


def _tfinal(a0, a1, s0, s1, bb, b3, Wr, br, Wm0, bm0, Wm1, bm1, Wl, bl):
    full = lambda i: (0, 0)
    return pl.pallas_call(
        _tf_body,
        grid=(NB,),
        in_specs=[
            pl.BlockSpec((BN, H), lambda i: (i, 0)),
            pl.BlockSpec((BN, H), lambda i: (i, 0)),
            pl.BlockSpec((BN, 1), lambda i: (i, 0)),
            pl.BlockSpec((BN, 1), lambda i: (i, 0)),
            pl.BlockSpec((BN, 1), lambda i: (i, 0)),
            pl.BlockSpec((1, H), full),
            pl.BlockSpec((2 * H, H), full),
            pl.BlockSpec((1, H), full),
            pl.BlockSpec((H, H), full),
            pl.BlockSpec((1, H), full),
            pl.BlockSpec((H, H), full),
            pl.BlockSpec((1, H), full),
            pl.BlockSpec((H, TV * FH), full),
            pl.BlockSpec((1, TV * FH), full),
        ],
        out_specs=pl.BlockSpec((G, TV * FH), full),
        out_shape=jax.ShapeDtypeStruct((G, TV * FH), jnp.float32),
        scratch_shapes=[
            pltpu.VMEM((G, H), jnp.float32),
            pltpu.VMEM((G, 1), jnp.float32),
            pltpu.VMEM((G, H), jnp.float32),
        ],
    )(a0, a1, s0, s1, bb, b3, Wr, br, Wm0, bm0, Wm1, bm1, Wl, bl)


def _sc_make():
    mesh = plsc.VectorSubcoreMesh(core_axis_name="c", subcore_axis_name="s",
                                  num_cores=2, num_subcores=16)

    @functools.partial(
        pl.kernel,
        out_type=[
            jax.ShapeDtypeStruct((2, NP, H), jnp.float32),
            jax.ShapeDtypeStruct((2, NP), jnp.float32),
        ],
        mesh=mesh,
        compiler_params=pltpu.CompilerParams(needs_layout_passes=False,
                                             use_tc_tiling_on_sc=False),
        scratch_types=[
            pltpu.VMEM((NP,), jnp.float32),
            pltpu.VMEM((NP,), jnp.float32),
            pltpu.VMEM((16,), jnp.float32),
            pltpu.VMEM((CPT, CE), jnp.int32),
            pltpu.VMEM((CPT, CE), jnp.int32),
            pltpu.VMEM((CE,), jnp.float32),
            pltpu.VMEM((CE,), jnp.float32),
            pltpu.VMEM((CE, H), jnp.float32),
            pltpu.VMEM((CE, H), jnp.float32),
            pltpu.VMEM((NP,), jnp.float32),
            pltpu.VMEM((ROWS_PT,), jnp.float32),
            pltpu.VMEM((ROWS_PT,), jnp.float32),
            pltpu.VMEM_SHARED((NP, H), jnp.float32),
            pltpu.VMEM_SHARED((16, NP), jnp.float32),
            pltpu.SemaphoreType.DMA,
            pltpu.SemaphoreType.DMA,
            pltpu.SemaphoreType.DMA,
            pltpu.SemaphoreType.DMA,
        ],
    )
    def sc_fn(als_hbm, ald_hbm, c_hbm, h_hbm, src_hbm, dst_hbm,
              out_hbm, s_hbm,
              als_v, ald_v, c_v, src_v, dst_v, ex0, ex1, rows0, rows1,
              s_v, z640, acc640, out_sh, slab, sem_g0, sem_g1, sem_s0, sem_s1):
        cid = lax.axis_index("c")
        tid = lax.axis_index("s")
        wid = tid * 2 + cid
        zero16 = jnp.zeros((16,), jnp.float32)

        def zrow(r, _):
            rows0[r, pl.ds(0, 16)] = zero16
            rows0[r, pl.ds(16, 16)] = zero16
            rows1[r, pl.ds(0, 16)] = zero16
            rows1[r, pl.ds(16, 16)] = zero16
            return 0

        lax.fori_loop(0, CE, zrow, 0)

        def zs(k, _):
            s_v[pl.ds(k * 16, 16)] = zero16
            return 0

        lax.fori_loop(0, NP // 16, zs, 0)

        def z6(k, _):
            z640[pl.ds(k * 16, 16)] = zero16
            acc640[pl.ds(k * 16, 16)] = zero16
            return 0

        lax.fori_loop(0, ROWS_PT // 16, z6, 0)

        pltpu.sync_copy(als_hbm, als_v)
        pltpu.sync_copy(ald_hbm, ald_v)
        pltpu.sync_copy(c_hbm, c_v)
        pltpu.sync_copy(src_hbm.at[wid], src_v)
        pltpu.sync_copy(dst_hbm.at[wid], dst_v)

        base = tid * ROWS_PT
        for k in range(ROWS_PT // CE):
            pltpu.sync_copy(rows0, out_sh.at[pl.ds(base + k * CE, CE)])
        plsc.subcore_barrier()

        pltpu.async_copy(rows0, out_sh.at[dst_v.at[0]], sem_s0, add=True)
        pltpu.async_copy(rows1, out_sh.at[dst_v.at[0]], sem_s1, add=True)

        cvec = c_v[...]

        def exphase(j, exbuf):
            for k in range(CE // 16):
                s16 = src_v[j, pl.ds(k * 16, 16)]
                d16 = dst_v[j, pl.ds(k * 16, 16)]
                av = plsc.load_gather(als_v, [s16])
                bv = plsc.load_gather(ald_v, [d16])
                e = av + bv
                e = jnp.maximum(e, 0.2 * e)
                ex = jnp.exp(e - cvec)
                exbuf[pl.ds(k * 16, 16)] = ex
                plsc.addupdate_scatter(s_v, [d16], ex)

        def scale(exbuf, rowsbuf):
            def srow(r, _):
                exr = plsc.load_gather(exbuf, [lax.broadcast(r, (16,))])
                rowsbuf[r, pl.ds(0, 16)] = rowsbuf[r, pl.ds(0, 16)] * exr
                rowsbuf[r, pl.ds(16, 16)] = rowsbuf[r, pl.ds(16, 16)] * exr
                return 0

            lax.fori_loop(0, CE, srow, 0)

        def pair(i, _):
            j0 = i * 2
            j1 = i * 2 + 1
            pltpu.make_async_copy(rows0, out_sh.at[dst_v.at[j0]], sem_s0).wait()
            pltpu.make_async_copy(rows1, out_sh.at[dst_v.at[j0]], sem_s1).wait()
            g0 = pltpu.async_copy(h_hbm.at[src_v.at[j0]], rows0, sem_g0)
            g1 = pltpu.async_copy(h_hbm.at[src_v.at[j1]], rows1, sem_g1)
            exphase(j0, ex0)
            g0.wait()
            scale(ex0, rows0)
            pltpu.async_copy(rows0, out_sh.at[dst_v.at[j0]], sem_s0, add=True)
            exphase(j1, ex1)
            g1.wait()
            scale(ex1, rows1)
            pltpu.async_copy(rows1, out_sh.at[dst_v.at[j1]], sem_s1, add=True)
            return 0

        lax.fori_loop(0, CPT // 2, pair, 0)
        pltpu.make_async_copy(rows0, out_sh.at[dst_v.at[0]], sem_s0).wait()
        pltpu.make_async_copy(rows1, out_sh.at[dst_v.at[0]], sem_s1).wait()

        pltpu.sync_copy(s_v, slab.at[tid])
        plsc.subcore_barrier()
        for u in range(16):
            pltpu.sync_copy(slab.at[u, pl.ds(base, ROWS_PT)], z640)

            def addv(k, _):
                acc640[pl.ds(k * 16, 16)] = (acc640[pl.ds(k * 16, 16)]
                                             + z640[pl.ds(k * 16, 16)])
                return 0

            lax.fori_loop(0, ROWS_PT // 16, addv, 0)
        pltpu.sync_copy(acc640, s_hbm.at[cid, pl.ds(base, ROWS_PT)])
        for k in range(ROWS_PT // CE):
            pltpu.sync_copy(out_sh.at[pl.ds(base + k * CE, CE)], rows0)
            pltpu.sync_copy(rows0, out_hbm.at[cid, pl.ds(base + k * CE, CE)])

    return sc_fn


_SC_CACHE = []


def _sc_fn():
    if not _SC_CACHE:
        _SC_CACHE.append(_sc_make())
    return _SC_CACHE[0]


def _sc_layer(h, als, ald, ma, mb, srcs, dsts):
    alsp = jnp.concatenate([als.reshape(N), jnp.full((NP - N,), -1e30, jnp.float32)])
    aldp = jnp.concatenate([ald.reshape(N), jnp.full((NP - N,), -1e30, jnp.float32)])
    hp = jnp.concatenate([h, jnp.zeros((NP - N, H), jnp.float32)], axis=0)
    m = ma[0, 0] + mb[0, 0]
    c = jnp.where(m >= 0.0, m, 0.2 * m)
    c16 = jnp.full((16,), c, jnp.float32)
    out_acc, s_out = _sc_fn()(alsp, aldp, c16, hp, srcs, dsts)
    a0 = out_acc[0, :N]
    a1 = out_acc[1, :N]
    s0 = s_out[0, :N].reshape(N, 1)
    s1 = s_out[1, :N].reshape(N, 1)
    return a0, a1, s0, s1


def kernel(x, edge_index, batch, W1, as1, ad1, b1, W2, as2, ad2, b2,
           W3, as3, ad3, b3, Wr, br, Wm0, bm0, Wm1, bm1, Wl, bl):
    loops = jnp.arange(N, dtype=edge_index.dtype)
    src = jnp.concatenate([edge_index[0], loops,
                           jnp.full((EPAD - EP,), N, edge_index.dtype)])
    dst = jnp.concatenate([edge_index[1], loops,
                           jnp.full((EPAD - EP,), N, edge_index.dtype)])
    srcs = src.reshape(NW, CPT, CE)
    dsts = dst.reshape(NW, CPT, CE)

    h1, als1, ald1, ma1, mb1 = _t1(x, W1, as1.reshape(H, 1), ad1.reshape(H, 1), D)
    a0, a1, s0, s1 = _sc_layer(h1, als1, ald1, ma1, mb1, srcs, dsts)

    h2, als2, ald2, ma2, mb2 = _tmid(a0, a1, s0, s1, b1.reshape(1, H), W2,
                                     as2.reshape(H, 1), ad2.reshape(H, 1))
    a0, a1, s0, s1 = _sc_layer(h2, als2, ald2, ma2, mb2, srcs, dsts)

    h3, als3, ald3, ma3, mb3 = _tmid(a0, a1, s0, s1, b2.reshape(1, H), W3,
                                     as3.reshape(H, 1), ad3.reshape(H, 1))
    a0, a1, s0, s1 = _sc_layer(h3, als3, ald3, ma3, mb3, srcs, dsts)

    z = _tfinal(a0, a1, s0, s1, batch.reshape(N, 1), b3.reshape(1, H),
                Wr, br.reshape(1, H), Wm0, bm0.reshape(1, H),
                Wm1, bm1.reshape(1, H), Wl, bl.reshape(1, TV * FH))
    return z.reshape(G, FH, TV)


# speedup vs baseline: 1.0343x; 1.0343x over previous
import functools

import jax
import jax.numpy as jnp
from jax import lax
from jax.experimental import pallas as pl
from jax.experimental.pallas import tpu as pltpu
from jax.experimental.pallas import tpu_sc as plsc

N = 10000
E = 320000
D = 128
H = 32
G = 64
FH = 12
TV = 4

NP = 10240
NW = 32
CE = 128
EP = E + N
CPT = 82
EPAD = NW * CPT * CE
ROWS_PT = NP // 16

NB = 5
BN = N // NB


def _t1_body(x_ref, w_ref, as_ref, ad_ref, h_ref, als_ref, ald_ref,
             ma_ref, mb_ref):
    i = pl.program_id(0)
    h = x_ref[...] @ w_ref[...]
    h_ref[...] = h
    als = h @ as_ref[...]
    ald = h @ ad_ref[...]
    als_ref[...] = als
    ald_ref[...] = ald

    @pl.when(i == 0)
    def _():
        ma_ref[...] = jnp.full((1, 1), -jnp.inf, jnp.float32)
        mb_ref[...] = jnp.full((1, 1), -jnp.inf, jnp.float32)

    ma_ref[...] = jnp.maximum(ma_ref[...], jnp.max(als))
    mb_ref[...] = jnp.maximum(mb_ref[...], jnp.max(ald))


def _t1(x, W, a_s, a_d, din):
    return pl.pallas_call(
        _t1_body,
        grid=(NB,),
        in_specs=[
            pl.BlockSpec((BN, din), lambda i: (i, 0)),
            pl.BlockSpec((din, H), lambda i: (0, 0)),
            pl.BlockSpec((H, 1), lambda i: (0, 0)),
            pl.BlockSpec((H, 1), lambda i: (0, 0)),
        ],
        out_specs=[
            pl.BlockSpec((BN, H), lambda i: (i, 0)),
            pl.BlockSpec((BN, 1), lambda i: (i, 0)),
            pl.BlockSpec((BN, 1), lambda i: (i, 0)),
            pl.BlockSpec((1, 1), lambda i: (0, 0)),
            pl.BlockSpec((1, 1), lambda i: (0, 0)),
        ],
        out_shape=[
            jax.ShapeDtypeStruct((N, H), jnp.float32),
            jax.ShapeDtypeStruct((N, 1), jnp.float32),
            jax.ShapeDtypeStruct((N, 1), jnp.float32),
            jax.ShapeDtypeStruct((1, 1), jnp.float32),
            jax.ShapeDtypeStruct((1, 1), jnp.float32),
        ],
    )(x, W, a_s, a_d)


def _tmid_body(a0_ref, a1_ref, s0_ref, s1_ref, b_ref, w_ref, as_ref, ad_ref,
               h_ref, als_ref, ald_ref, ma_ref, mb_ref):
    i = pl.program_id(0)
    pre = (a0_ref[...] + a1_ref[...]) / (s0_ref[...] + s1_ref[...] + 1e-16)
    pre = pre + b_ref[...]
    act = 0.5 * pre * (1.0 + lax.erf(pre * (2.0 ** -0.5)))
    h = act @ w_ref[...]
    h_ref[...] = h
    als = h @ as_ref[...]
    ald = h @ ad_ref[...]
    als_ref[...] = als
    ald_ref[...] = ald

    @pl.when(i == 0)
    def _():
        ma_ref[...] = jnp.full((1, 1), -jnp.inf, jnp.float32)
        mb_ref[...] = jnp.full((1, 1), -jnp.inf, jnp.float32)

    ma_ref[...] = jnp.maximum(ma_ref[...], jnp.max(als))
    mb_ref[...] = jnp.maximum(mb_ref[...], jnp.max(ald))


def _tmid(a0, a1, s0, s1, b, W, a_s, a_d):
    return pl.pallas_call(
        _tmid_body,
        grid=(NB,),
        in_specs=[
            pl.BlockSpec((BN, H), lambda i: (i, 0)),
            pl.BlockSpec((BN, H), lambda i: (i, 0)),
            pl.BlockSpec((BN, 1), lambda i: (i, 0)),
            pl.BlockSpec((BN, 1), lambda i: (i, 0)),
            pl.BlockSpec((1, H), lambda i: (0, 0)),
            pl.BlockSpec((H, H), lambda i: (0, 0)),
            pl.BlockSpec((H, 1), lambda i: (0, 0)),
            pl.BlockSpec((H, 1), lambda i: (0, 0)),
        ],
        out_specs=[
            pl.BlockSpec((BN, H), lambda i: (i, 0)),
            pl.BlockSpec((BN, 1), lambda i: (i, 0)),
            pl.BlockSpec((BN, 1), lambda i: (i, 0)),
            pl.BlockSpec((1, 1), lambda i: (0, 0)),
            pl.BlockSpec((1, 1), lambda i: (0, 0)),
        ],
        out_shape=[
            jax.ShapeDtypeStruct((N, H), jnp.float32),
            jax.ShapeDtypeStruct((N, 1), jnp.float32),
            jax.ShapeDtypeStruct((N, 1), jnp.float32),
            jax.ShapeDtypeStruct((1, 1), jnp.float32),
            jax.ShapeDtypeStruct((1, 1), jnp.float32),
        ],
    )(a0, a1, s0, s1, b, W, a_s, a_d)


def _tf_body(a0_ref, a1_ref, s0_ref, s1_ref, bb_ref, b3_ref, wr_ref, br_ref,
             wm0_ref, bm0_ref, wm1_ref, bm1_ref, wl_ref, bl_ref,
             out_ref, sum_s, cnt_s, mx_s):
    i = pl.program_id(0)

    @pl.when(i == 0)
    def _():
        sum_s[...] = jnp.zeros((G, H), jnp.float32)
        cnt_s[...] = jnp.zeros((G, 1), jnp.float32)
        mx_s[...] = jnp.full((G, H), -jnp.inf, jnp.float32)

    h = (a0_ref[...] + a1_ref[...]) / (s0_ref[...] + s1_ref[...] + 1e-16)
    h = h + b3_ref[...]
    bb = bb_ref[...]
    oh = (bb == lax.broadcasted_iota(jnp.int32, (1, G), 1)).astype(jnp.float32)
    sum_s[...] += lax.dot_general(oh, h, (((0,), (0,)), ((), ())))
    cnt_s[...] += jnp.sum(oh, axis=0)[:, None]
    for g in range(G):
        mg = jnp.where(bb == g, h, -jnp.inf)
        mx_s[g:g + 1, :] = jnp.maximum(mx_s[g:g + 1, :],
                                       jnp.max(mg, axis=0, keepdims=True))

    @pl.when(i == NB - 1)
    def _():
        mean = sum_s[...] / jnp.maximum(cnt_s[...], 1.0)
        mxf = jnp.where(jnp.isfinite(mx_s[...]), mx_s[...], 0.0)
        z = jnp.concatenate([mean, mxf], axis=1)
        z = z @ wr_ref[...] + br_ref[...]
        z = jnp.maximum(z @ wm0_ref[...] + bm0_ref[...], 0.0)
        z = jnp.maximum(z @ wm1_ref[...] + bm1_ref[...], 0.0)
        out_ref[...] = z @ wl_ref[...] + bl_ref[...]


def _tfinal(a0, a1, s0, s1, bb, b3, Wr, br, Wm0, bm0, Wm1, bm1, Wl, bl):
    full = lambda i: (0, 0)
    return pl.pallas_call(
        _tf_body,
        grid=(NB,),
        in_specs=[
            pl.BlockSpec((BN, H), lambda i: (i, 0)),
            pl.BlockSpec((BN, H), lambda i: (i, 0)),
            pl.BlockSpec((BN, 1), lambda i: (i, 0)),
            pl.BlockSpec((BN, 1), lambda i: (i, 0)),
            pl.BlockSpec((BN, 1), lambda i: (i, 0)),
            pl.BlockSpec((1, H), full),
            pl.BlockSpec((2 * H, H), full),
            pl.BlockSpec((1, H), full),
            pl.BlockSpec((H, H), full),
            pl.BlockSpec((1, H), full),
            pl.BlockSpec((H, H), full),
            pl.BlockSpec((1, H), full),
            pl.BlockSpec((H, TV * FH), full),
            pl.BlockSpec((1, TV * FH), full),
        ],
        out_specs=pl.BlockSpec((G, TV * FH), full),
        out_shape=jax.ShapeDtypeStruct((G, TV * FH), jnp.float32),
        scratch_shapes=[
            pltpu.VMEM((G, H), jnp.float32),
            pltpu.VMEM((G, 1), jnp.float32),
            pltpu.VMEM((G, H), jnp.float32),
        ],
    )(a0, a1, s0, s1, bb, b3, Wr, br, Wm0, bm0, Wm1, bm1, Wl, bl)


def _sc_make():
    mesh = plsc.VectorSubcoreMesh(core_axis_name="c", subcore_axis_name="s",
                                  num_cores=2, num_subcores=16)

    @functools.partial(
        pl.kernel,
        out_type=[
            jax.ShapeDtypeStruct((2, NP, H), jnp.float32),
            jax.ShapeDtypeStruct((2, NP), jnp.float32),
        ],
        mesh=mesh,
        compiler_params=pltpu.CompilerParams(needs_layout_passes=False,
                                             use_tc_tiling_on_sc=False),
        scratch_types=[
            pltpu.VMEM((NP,), jnp.float32),
            pltpu.VMEM((NP,), jnp.float32),
            pltpu.VMEM((16,), jnp.float32),
            pltpu.VMEM((CPT, CE), jnp.int32),
            pltpu.VMEM((CPT, CE), jnp.int32),
            pltpu.VMEM((CE,), jnp.float32),
            pltpu.VMEM((CE,), jnp.float32),
            pltpu.VMEM((CE, H), jnp.float32),
            pltpu.VMEM((CE, H), jnp.float32),
            pltpu.VMEM((ROWS_PT,), jnp.float32),
            pltpu.VMEM_SHARED((NP, H), jnp.float32),
            pltpu.VMEM_SHARED((NP,), jnp.float32),
            pltpu.SemaphoreType.DMA,
            pltpu.SemaphoreType.DMA,
            pltpu.SemaphoreType.DMA,
            pltpu.SemaphoreType.DMA,
            pltpu.SemaphoreType.DMA,
            pltpu.SemaphoreType.DMA,
        ],
    )
    def sc_fn(als_hbm, ald_hbm, c_hbm, h_hbm, src_hbm, dst_hbm,
              out_hbm, s_hbm,
              als_v, ald_v, c_v, src_v, dst_v, ex0, ex1, rows0, rows1,
              z640, out_sh, s_sh, sem_g0, sem_g1, sem_s0, sem_s1,
              sem_e0, sem_e1):
        cid = lax.axis_index("c")
        tid = lax.axis_index("s")
        wid = tid * 2 + cid
        zero16 = jnp.zeros((16,), jnp.float32)

        def zrow(r, _):
            rows0[r, pl.ds(0, 16)] = zero16
            rows0[r, pl.ds(16, 16)] = zero16
            rows1[r, pl.ds(0, 16)] = zero16
            rows1[r, pl.ds(16, 16)] = zero16
            return 0

        lax.fori_loop(0, CE, zrow, 0)

        def zex(k, _):
            ex0[pl.ds(k * 16, 16)] = zero16
            ex1[pl.ds(k * 16, 16)] = zero16
            return 0

        lax.fori_loop(0, CE // 16, zex, 0)

        def z6(k, _):
            z640[pl.ds(k * 16, 16)] = zero16
            return 0

        lax.fori_loop(0, ROWS_PT // 16, z6, 0)

        pltpu.sync_copy(als_hbm, als_v)
        pltpu.sync_copy(ald_hbm, ald_v)
        pltpu.sync_copy(c_hbm, c_v)
        pltpu.sync_copy(src_hbm.at[wid], src_v)
        pltpu.sync_copy(dst_hbm.at[wid], dst_v)

        base = tid * ROWS_PT
        pltpu.sync_copy(z640, s_sh.at[pl.ds(base, ROWS_PT)])
        for k in range(ROWS_PT // CE):
            pltpu.sync_copy(rows0, out_sh.at[pl.ds(base + k * CE, CE)])
        plsc.subcore_barrier()

        pltpu.async_copy(rows0, out_sh.at[dst_v.at[0]], sem_s0, add=True)
        pltpu.async_copy(rows1, out_sh.at[dst_v.at[0]], sem_s1, add=True)
        pltpu.async_copy(ex0, s_sh.at[dst_v.at[0]], sem_e0, add=True)
        pltpu.async_copy(ex1, s_sh.at[dst_v.at[0]], sem_e1, add=True)

        cvec = c_v[...]

        def exphase(j, exbuf):
            for k in range(CE // 16):
                s16 = src_v[j, pl.ds(k * 16, 16)]
                d16 = dst_v[j, pl.ds(k * 16, 16)]
                av = plsc.load_gather(als_v, [s16])
                bv = plsc.load_gather(ald_v, [d16])
                e = av + bv
                e = jnp.maximum(e, 0.2 * e)
                exbuf[pl.ds(k * 16, 16)] = jnp.exp(e - cvec)

        def scale(exbuf, rowsbuf):
            def srow(r, _):
                exr = plsc.load_gather(exbuf, [lax.broadcast(r, (16,))])
                rowsbuf[r, pl.ds(0, 16)] = rowsbuf[r, pl.ds(0, 16)] * exr
                rowsbuf[r, pl.ds(16, 16)] = rowsbuf[r, pl.ds(16, 16)] * exr
                return 0

            lax.fori_loop(0, CE, srow, 0)

        def pair(i, _):
            j0 = i * 2
            j1 = i * 2 + 1
            pltpu.make_async_copy(rows0, out_sh.at[dst_v.at[j0]], sem_s0).wait()
            pltpu.make_async_copy(rows1, out_sh.at[dst_v.at[j0]], sem_s1).wait()
            pltpu.make_async_copy(ex0, s_sh.at[dst_v.at[j0]], sem_e0).wait()
            pltpu.make_async_copy(ex1, s_sh.at[dst_v.at[j0]], sem_e1).wait()
            g0 = pltpu.async_copy(h_hbm.at[src_v.at[j0]], rows0, sem_g0)
            g1 = pltpu.async_copy(h_hbm.at[src_v.at[j1]], rows1, sem_g1)
            exphase(j0, ex0)
            pltpu.async_copy(ex0, s_sh.at[dst_v.at[j0]], sem_e0, add=True)
            g0.wait()
            scale(ex0, rows0)
            pltpu.async_copy(rows0, out_sh.at[dst_v.at[j0]], sem_s0, add=True)
            exphase(j1, ex1)
            pltpu.async_copy(ex1, s_sh.at[dst_v.at[j1]], sem_e1, add=True)
            g1.wait()
            scale(ex1, rows1)
            pltpu.async_copy(rows1, out_sh.at[dst_v.at[j1]], sem_s1, add=True)
            return 0

        lax.fori_loop(0, CPT // 2, pair, 0)
        pltpu.make_async_copy(rows0, out_sh.at[dst_v.at[0]], sem_s0).wait()
        pltpu.make_async_copy(rows1, out_sh.at[dst_v.at[0]], sem_s1).wait()
        pltpu.make_async_copy(ex0, s_sh.at[dst_v.at[0]], sem_e0).wait()
        pltpu.make_async_copy(ex1, s_sh.at[dst_v.at[0]], sem_e1).wait()

        plsc.subcore_barrier()
        pltpu.sync_copy(s_sh.at[pl.ds(base, ROWS_PT)], z640)
        pltpu.sync_copy(z640, s_hbm.at[cid, pl.ds(base, ROWS_PT)])
        for k in range(ROWS_PT // CE):
            pltpu.sync_copy(out_sh.at[pl.ds(base + k * CE, CE)], rows0)
            pltpu.sync_copy(rows0, out_hbm.at[cid, pl.ds(base + k * CE, CE)])

    return sc_fn


_SC_CACHE = []


def _sc_fn():
    if not _SC_CACHE:
        _SC_CACHE.append(_sc_make())
    return _SC_CACHE[0]


def _sc_layer(h, als, ald, ma, mb, srcs, dsts):
    alsp = jnp.concatenate([als.reshape(N), jnp.full((NP - N,), -1e30, jnp.float32)])
    aldp = jnp.concatenate([ald.reshape(N), jnp.full((NP - N,), -1e30, jnp.float32)])
    hp = jnp.concatenate([h, jnp.zeros((NP - N, H), jnp.float32)], axis=0)
    m = ma[0, 0] + mb[0, 0]
    c = jnp.where(m >= 0.0, m, 0.2 * m)
    c16 = jnp.full((16,), c, jnp.float32)
    out_acc, s_out = _sc_fn()(alsp, aldp, c16, hp, srcs, dsts)
    a0 = out_acc[0, :N]
    a1 = out_acc[1, :N]
    s0 = s_out[0, :N].reshape(N, 1)
    s1 = s_out[1, :N].reshape(N, 1)
    return a0, a1, s0, s1


def kernel(x, edge_index, batch, W1, as1, ad1, b1, W2, as2, ad2, b2,
           W3, as3, ad3, b3, Wr, br, Wm0, bm0, Wm1, bm1, Wl, bl):
    loops = jnp.arange(N, dtype=edge_index.dtype)
    src = jnp.concatenate([edge_index[0], loops,
                           jnp.full((EPAD - EP,), N, edge_index.dtype)])
    dst = jnp.concatenate([edge_index[1], loops,
                           jnp.full((EPAD - EP,), N, edge_index.dtype)])
    srcs = src.reshape(NW, CPT, CE)
    dsts = dst.reshape(NW, CPT, CE)

    h1, als1, ald1, ma1, mb1 = _t1(x, W1, as1.reshape(H, 1), ad1.reshape(H, 1), D)
    a0, a1, s0, s1 = _sc_layer(h1, als1, ald1, ma1, mb1, srcs, dsts)

    h2, als2, ald2, ma2, mb2 = _tmid(a0, a1, s0, s1, b1.reshape(1, H), W2,
                                     as2.reshape(H, 1), ad2.reshape(H, 1))
    a0, a1, s0, s1 = _sc_layer(h2, als2, ald2, ma2, mb2, srcs, dsts)

    h3, als3, ald3, ma3, mb3 = _tmid(a0, a1, s0, s1, b2.reshape(1, H), W3,
                                     as3.reshape(H, 1), ad3.reshape(H, 1))
    a0, a1, s0, s1 = _sc_layer(h3, als3, ald3, ma3, mb3, srcs, dsts)

    z = _tfinal(a0, a1, s0, s1, batch.reshape(N, 1), b3.reshape(1, H),
                Wr, br.reshape(1, H), Wm0, bm0.reshape(1, H),
                Wm1, bm1.reshape(1, H), Wl, bl.reshape(1, TV * FH))
    return z.reshape(G, FH, TV)


# speedup vs baseline: 1.1592x; 1.1208x over previous
import functools

import jax
import jax.numpy as jnp
from jax import lax
from jax.experimental import pallas as pl
from jax.experimental.pallas import tpu as pltpu
from jax.experimental.pallas import tpu_sc as plsc

N = 10000
E = 320000
D = 128
H = 32
G = 64
FH = 12
TV = 4

NP = 10240
NW = 32
CE = 128
EP = E + N
CPT = -(-EP // (NW * CE))
EPAD = NW * CPT * CE
ROWS_PT = NP // 16

NB = 10
BNP = NP // NB

_NEG = -1e30


def _t1_body(x_ref, w_ref, as_ref, ad_ref, h_ref, als_ref, ald_ref,
             ma_ref, mb_ref):
    i = pl.program_id(0)
    h = x_ref[...] @ w_ref[...]
    h_ref[...] = h
    als = h @ as_ref[...]
    ald = h @ ad_ref[...]
    rid = i * BNP + lax.broadcasted_iota(jnp.int32, (BNP, 1), 0)
    mask = rid < N
    als_ref[...] = jnp.where(mask, als, _NEG)
    ald_ref[...] = jnp.where(mask, ald, _NEG)

    @pl.when(i == 0)
    def _():
        ma_ref[...] = jnp.full((1, 1), -jnp.inf, jnp.float32)
        mb_ref[...] = jnp.full((1, 1), -jnp.inf, jnp.float32)

    ma_ref[...] = jnp.maximum(ma_ref[...], jnp.max(jnp.where(mask, als, -jnp.inf)))
    mb_ref[...] = jnp.maximum(mb_ref[...], jnp.max(jnp.where(mask, ald, -jnp.inf)))


def _t1(x, W, a_s, a_d):
    return pl.pallas_call(
        _t1_body,
        grid=(NB,),
        in_specs=[
            pl.BlockSpec((BNP, D), lambda i: (i, 0)),
            pl.BlockSpec((D, H), lambda i: (0, 0)),
            pl.BlockSpec((H, 1), lambda i: (0, 0)),
            pl.BlockSpec((H, 1), lambda i: (0, 0)),
        ],
        out_specs=[
            pl.BlockSpec((BNP, H), lambda i: (i, 0)),
            pl.BlockSpec((BNP, 1), lambda i: (i, 0)),
            pl.BlockSpec((BNP, 1), lambda i: (i, 0)),
            pl.BlockSpec((1, 1), lambda i: (0, 0)),
            pl.BlockSpec((1, 1), lambda i: (0, 0)),
        ],
        out_shape=[
            jax.ShapeDtypeStruct((NP, H), jnp.float32),
            jax.ShapeDtypeStruct((NP, 1), jnp.float32),
            jax.ShapeDtypeStruct((NP, 1), jnp.float32),
            jax.ShapeDtypeStruct((1, 1), jnp.float32),
            jax.ShapeDtypeStruct((1, 1), jnp.float32),
        ],
    )(x, W, a_s, a_d)


def _tmid_body(a_ref, s_ref, b_ref, w_ref, as_ref, ad_ref,
               h_ref, als_ref, ald_ref, ma_ref, mb_ref):
    i = pl.program_id(0)
    a = a_ref[...]
    s = s_ref[...]
    denom = (s[0] + s[1] + 1e-16)[:, None]
    pre = (a[0] + a[1]) / denom + b_ref[...]
    act = 0.5 * pre * (1.0 + lax.erf(pre * (2.0 ** -0.5)))
    h = act @ w_ref[...]
    h_ref[...] = h
    als = h @ as_ref[...]
    ald = h @ ad_ref[...]
    rid = i * BNP + lax.broadcasted_iota(jnp.int32, (BNP, 1), 0)
    mask = rid < N
    als_ref[...] = jnp.where(mask, als, _NEG)
    ald_ref[...] = jnp.where(mask, ald, _NEG)

    @pl.when(i == 0)
    def _():
        ma_ref[...] = jnp.full((1, 1), -jnp.inf, jnp.float32)
        mb_ref[...] = jnp.full((1, 1), -jnp.inf, jnp.float32)

    ma_ref[...] = jnp.maximum(ma_ref[...], jnp.max(jnp.where(mask, als, -jnp.inf)))
    mb_ref[...] = jnp.maximum(mb_ref[...], jnp.max(jnp.where(mask, ald, -jnp.inf)))


def _tmid(acc, s, b, W, a_s, a_d):
    return pl.pallas_call(
        _tmid_body,
        grid=(NB,),
        in_specs=[
            pl.BlockSpec((2, BNP, H), lambda i: (0, i, 0)),
            pl.BlockSpec((2, BNP), lambda i: (0, i)),
            pl.BlockSpec((1, H), lambda i: (0, 0)),
            pl.BlockSpec((H, H), lambda i: (0, 0)),
            pl.BlockSpec((H, 1), lambda i: (0, 0)),
            pl.BlockSpec((H, 1), lambda i: (0, 0)),
        ],
        out_specs=[
            pl.BlockSpec((BNP, H), lambda i: (i, 0)),
            pl.BlockSpec((BNP, 1), lambda i: (i, 0)),
            pl.BlockSpec((BNP, 1), lambda i: (i, 0)),
            pl.BlockSpec((1, 1), lambda i: (0, 0)),
            pl.BlockSpec((1, 1), lambda i: (0, 0)),
        ],
        out_shape=[
            jax.ShapeDtypeStruct((NP, H), jnp.float32),
            jax.ShapeDtypeStruct((NP, 1), jnp.float32),
            jax.ShapeDtypeStruct((NP, 1), jnp.float32),
            jax.ShapeDtypeStruct((1, 1), jnp.float32),
            jax.ShapeDtypeStruct((1, 1), jnp.float32),
        ],
    )(acc, s, b, W, a_s, a_d)


def _tf_body(a_ref, s_ref, bb_ref, b3_ref, wr_ref, br_ref,
             wm0_ref, bm0_ref, wm1_ref, bm1_ref, wl_ref, bl_ref,
             out_ref, sum_s, cnt_s, mx_s):
    i = pl.program_id(0)

    @pl.when(i == 0)
    def _():
        sum_s[...] = jnp.zeros((G, H), jnp.float32)
        cnt_s[...] = jnp.zeros((G, 1), jnp.float32)
        mx_s[...] = jnp.full((G, H), -jnp.inf, jnp.float32)

    a = a_ref[...]
    s = s_ref[...]
    denom = (s[0] + s[1] + 1e-16)[:, None]
    h = (a[0] + a[1]) / denom + b3_ref[...]
    bb = bb_ref[...]
    oh = (bb == lax.broadcasted_iota(jnp.int32, (1, G), 1)).astype(jnp.float32)
    sum_s[...] += lax.dot_general(oh, h, (((0,), (0,)), ((), ())))
    cnt_s[...] += jnp.sum(oh, axis=0)[:, None]
    for g in range(G):
        mg = jnp.where(bb == g, h, -jnp.inf)
        mx_s[g:g + 1, :] = jnp.maximum(mx_s[g:g + 1, :],
                                       jnp.max(mg, axis=0, keepdims=True))

    @pl.when(i == NB - 1)
    def _():
        mean = sum_s[...] / jnp.maximum(cnt_s[...], 1.0)
        mxf = jnp.where(jnp.isfinite(mx_s[...]), mx_s[...], 0.0)
        z = jnp.concatenate([mean, mxf], axis=1)
        z = z @ wr_ref[...] + br_ref[...]
        z = jnp.maximum(z @ wm0_ref[...] + bm0_ref[...], 0.0)
        z = jnp.maximum(z @ wm1_ref[...] + bm1_ref[...], 0.0)
        out_ref[...] = z @ wl_ref[...] + bl_ref[...]


def _tfinal(acc, s, bb, b3, Wr, br, Wm0, bm0, Wm1, bm1, Wl, bl):
    full = lambda i: (0, 0)
    return pl.pallas_call(
        _tf_body,
        grid=(NB,),
        in_specs=[
            pl.BlockSpec((2, BNP, H), lambda i: (0, i, 0)),
            pl.BlockSpec((2, BNP), lambda i: (0, i)),
            pl.BlockSpec((BNP, 1), lambda i: (i, 0)),
            pl.BlockSpec((1, H), full),
            pl.BlockSpec((2 * H, H), full),
            pl.BlockSpec((1, H), full),
            pl.BlockSpec((H, H), full),
            pl.BlockSpec((1, H), full),
            pl.BlockSpec((H, H), full),
            pl.BlockSpec((1, H), full),
            pl.BlockSpec((H, TV * FH), full),
            pl.BlockSpec((1, TV * FH), full),
        ],
        out_specs=pl.BlockSpec((G, TV * FH), full),
        out_shape=jax.ShapeDtypeStruct((G, TV * FH), jnp.float32),
        scratch_shapes=[
            pltpu.VMEM((G, H), jnp.float32),
            pltpu.VMEM((G, 1), jnp.float32),
            pltpu.VMEM((G, H), jnp.float32),
        ],
    )(acc, s, bb, b3, Wr, br, Wm0, bm0, Wm1, bm1, Wl, bl)


def _sc_make():
    mesh = plsc.VectorSubcoreMesh(core_axis_name="c", subcore_axis_name="s",
                                  num_cores=2, num_subcores=16)

    @functools.partial(
        pl.kernel,
        out_type=[
            jax.ShapeDtypeStruct((2, NP, H), jnp.float32),
            jax.ShapeDtypeStruct((2, NP), jnp.float32),
        ],
        mesh=mesh,
        compiler_params=pltpu.CompilerParams(needs_layout_passes=False,
                                             use_tc_tiling_on_sc=False),
        scratch_types=[
            pltpu.VMEM((NP,), jnp.float32),
            pltpu.VMEM((NP,), jnp.float32),
            pltpu.VMEM((16,), jnp.float32),
            pltpu.VMEM((CPT, CE), jnp.int32),
            pltpu.VMEM((CPT, CE), jnp.int32),
            pltpu.VMEM((CE,), jnp.float32),
            pltpu.VMEM((CE, H), jnp.float32),
            pltpu.VMEM((ROWS_PT,), jnp.float32),
            pltpu.VMEM_SHARED((NP, H), jnp.float32),
            pltpu.VMEM_SHARED((NP,), jnp.float32),
        ],
    )
    def sc_fn(als_hbm, ald_hbm, c_hbm, h_hbm, src_hbm, dst_hbm,
              out_hbm, s_hbm,
              als_v, ald_v, c_v, src_v, dst_v, ex_v, rows_v, z640,
              out_sh, s_sh):
        cid = lax.axis_index("c")
        tid = lax.axis_index("s")
        wid = tid * 2 + cid
        zero16 = jnp.zeros((16,), jnp.float32)

        def zrow(r, _):
            rows_v[r, pl.ds(0, 16)] = zero16
            rows_v[r, pl.ds(16, 16)] = zero16
            return 0

        lax.fori_loop(0, CE, zrow, 0)

        def z6(k, _):
            z640[pl.ds(k * 16, 16)] = zero16
            return 0

        lax.fori_loop(0, ROWS_PT // 16, z6, 0)

        pltpu.sync_copy(als_hbm, als_v)
        pltpu.sync_copy(ald_hbm, ald_v)
        pltpu.sync_copy(c_hbm, c_v)
        pltpu.sync_copy(src_hbm.at[wid], src_v)
        pltpu.sync_copy(dst_hbm.at[wid], dst_v)

        base = tid * ROWS_PT
        pltpu.sync_copy(z640, s_sh.at[pl.ds(base, ROWS_PT)])
        for k in range(ROWS_PT // CE):
            pltpu.sync_copy(rows_v, out_sh.at[pl.ds(base + k * CE, CE)])
        plsc.subcore_barrier()

        cvec = c_v[...]

        def chunk(j, _):
            for k in range(CE // 16):
                s16 = src_v[j, pl.ds(k * 16, 16)]
                d16 = dst_v[j, pl.ds(k * 16, 16)]
                av = plsc.load_gather(als_v, [s16])
                bv = plsc.load_gather(ald_v, [d16])
                e = av + bv
                e = jnp.maximum(e, 0.2 * e)
                ex_v[pl.ds(k * 16, 16)] = jnp.exp(e - cvec)

            pltpu.sync_copy(h_hbm.at[src_v.at[j]], rows_v)

            def srow(r, _):
                exr = plsc.load_gather(ex_v, [lax.broadcast(r, (16,))])
                rows_v[r, pl.ds(0, 16)] = rows_v[r, pl.ds(0, 16)] * exr
                rows_v[r, pl.ds(16, 16)] = rows_v[r, pl.ds(16, 16)] * exr
                return 0

            lax.fori_loop(0, CE, srow, 0)

            pltpu.sync_copy(ex_v, s_sh.at[dst_v.at[j]], add=True)
            pltpu.sync_copy(rows_v, out_sh.at[dst_v.at[j]], add=True)
            return 0

        lax.fori_loop(0, CPT, chunk, 0)
        plsc.subcore_barrier()

        pltpu.sync_copy(s_sh.at[pl.ds(base, ROWS_PT)], z640)
        pltpu.sync_copy(z640, s_hbm.at[cid, pl.ds(base, ROWS_PT)])
        for k in range(ROWS_PT // CE):
            pltpu.sync_copy(out_sh.at[pl.ds(base + k * CE, CE)], rows_v)
            pltpu.sync_copy(rows_v, out_hbm.at[cid, pl.ds(base + k * CE, CE)])

    return sc_fn


_SC_CACHE = []


def _sc_fn():
    if not _SC_CACHE:
        _SC_CACHE.append(_sc_make())
    return _SC_CACHE[0]


def _sc_layer(h, als, ald, ma, mb, srcs, dsts):
    m = ma[0, 0] + mb[0, 0]
    c = jnp.where(m >= 0.0, m, 0.2 * m)
    c16 = jnp.full((16,), c, jnp.float32)
    return _sc_fn()(als.reshape(NP), ald.reshape(NP), c16, h, srcs, dsts)


def kernel(x, edge_index, batch, W1, as1, ad1, b1, W2, as2, ad2, b2,
           W3, as3, ad3, b3, Wr, br, Wm0, bm0, Wm1, bm1, Wl, bl):
    loops = jnp.arange(N, dtype=edge_index.dtype)
    src = jnp.concatenate([edge_index[0], loops,
                           jnp.full((EPAD - EP,), N, edge_index.dtype)])
    dst = jnp.concatenate([edge_index[1], loops,
                           jnp.full((EPAD - EP,), N, edge_index.dtype)])
    srcs = src.reshape(NW, CPT, CE)
    dsts = dst.reshape(NW, CPT, CE)
    x_p = jnp.concatenate([x, jnp.zeros((NP - N, D), jnp.float32)], axis=0)
    batch_p = jnp.concatenate([batch, jnp.full((NP - N,), G, batch.dtype)])
    batch_p = batch_p.reshape(NP, 1)

    h1, als1, ald1, ma1, mb1 = _t1(x_p, W1, as1.reshape(H, 1), ad1.reshape(H, 1))
    acc, s = _sc_layer(h1, als1, ald1, ma1, mb1, srcs, dsts)

    h2, als2, ald2, ma2, mb2 = _tmid(acc, s, b1.reshape(1, H), W2,
                                     as2.reshape(H, 1), ad2.reshape(H, 1))
    acc, s = _sc_layer(h2, als2, ald2, ma2, mb2, srcs, dsts)

    h3, als3, ald3, ma3, mb3 = _tmid(acc, s, b2.reshape(1, H), W3,
                                     as3.reshape(H, 1), ad3.reshape(H, 1))
    acc, s = _sc_layer(h3, als3, ald3, ma3, mb3, srcs, dsts)

    z = _tfinal(acc, s, batch_p, b3.reshape(1, H),
                Wr, br.reshape(1, H), Wm0, bm0.reshape(1, H),
                Wm1, bm1.reshape(1, H), Wl, bl.reshape(1, TV * FH))
    return z.reshape(G, FH, TV)


# speedup vs baseline: 1.2653x; 1.0915x over previous
import functools

import jax
import jax.numpy as jnp
from jax import lax
from jax.experimental import pallas as pl
from jax.experimental.pallas import tpu as pltpu
from jax.experimental.pallas import tpu_sc as plsc

N = 10000
E = 320000
D = 128
H = 32
G = 64
FH = 12
TV = 4

NP = 10240
NW = 32
CE = 128
EP = E + N
CPT = -(-EP // (NW * CE))
EPAD = NW * CPT * CE
ROWS_PT = NP // 16

NB = 10
BNP = NP // NB

_NEG = -1e30


def _t1_body(x_ref, w_ref, as_ref, ad_ref, h_ref, als_ref, ald_ref,
             ma_ref, mb_ref):
    i = pl.program_id(0)
    h = x_ref[...] @ w_ref[...]
    h_ref[...] = h
    als = h @ as_ref[...]
    ald = h @ ad_ref[...]
    rid = i * BNP + lax.broadcasted_iota(jnp.int32, (BNP, 1), 0)
    mask = rid < N
    als_ref[...] = jnp.where(mask, als, _NEG)
    ald_ref[...] = jnp.where(mask, ald, _NEG)

    @pl.when(i == 0)
    def _():
        ma_ref[...] = jnp.full((1, 1), -jnp.inf, jnp.float32)
        mb_ref[...] = jnp.full((1, 1), -jnp.inf, jnp.float32)

    ma_ref[...] = jnp.maximum(ma_ref[...], jnp.max(jnp.where(mask, als, -jnp.inf)))
    mb_ref[...] = jnp.maximum(mb_ref[...], jnp.max(jnp.where(mask, ald, -jnp.inf)))


def _t1(x, W, a_s, a_d):
    return pl.pallas_call(
        _t1_body,
        grid=(NB,),
        in_specs=[
            pl.BlockSpec((BNP, D), lambda i: (i, 0)),
            pl.BlockSpec((D, H), lambda i: (0, 0)),
            pl.BlockSpec((H, 1), lambda i: (0, 0)),
            pl.BlockSpec((H, 1), lambda i: (0, 0)),
        ],
        out_specs=[
            pl.BlockSpec((BNP, H), lambda i: (i, 0)),
            pl.BlockSpec((BNP, 1), lambda i: (i, 0)),
            pl.BlockSpec((BNP, 1), lambda i: (i, 0)),
            pl.BlockSpec((1, 1), lambda i: (0, 0)),
            pl.BlockSpec((1, 1), lambda i: (0, 0)),
        ],
        out_shape=[
            jax.ShapeDtypeStruct((NP, H), jnp.float32),
            jax.ShapeDtypeStruct((NP, 1), jnp.float32),
            jax.ShapeDtypeStruct((NP, 1), jnp.float32),
            jax.ShapeDtypeStruct((1, 1), jnp.float32),
            jax.ShapeDtypeStruct((1, 1), jnp.float32),
        ],
    )(x, W, a_s, a_d)


def _tmid_body(a_ref, s_ref, b_ref, w_ref, as_ref, ad_ref,
               h_ref, als_ref, ald_ref, ma_ref, mb_ref):
    i = pl.program_id(0)
    a = a_ref[...]
    s = s_ref[...]
    denom = (s[0] + s[1] + 1e-16)[:, None]
    pre = (a[0] + a[1]) / denom + b_ref[...]
    act = 0.5 * pre * (1.0 + lax.erf(pre * (2.0 ** -0.5)))
    h = act @ w_ref[...]
    h_ref[...] = h
    als = h @ as_ref[...]
    ald = h @ ad_ref[...]
    rid = i * BNP + lax.broadcasted_iota(jnp.int32, (BNP, 1), 0)
    mask = rid < N
    als_ref[...] = jnp.where(mask, als, _NEG)
    ald_ref[...] = jnp.where(mask, ald, _NEG)

    @pl.when(i == 0)
    def _():
        ma_ref[...] = jnp.full((1, 1), -jnp.inf, jnp.float32)
        mb_ref[...] = jnp.full((1, 1), -jnp.inf, jnp.float32)

    ma_ref[...] = jnp.maximum(ma_ref[...], jnp.max(jnp.where(mask, als, -jnp.inf)))
    mb_ref[...] = jnp.maximum(mb_ref[...], jnp.max(jnp.where(mask, ald, -jnp.inf)))


def _tmid(acc, s, b, W, a_s, a_d):
    return pl.pallas_call(
        _tmid_body,
        grid=(NB,),
        in_specs=[
            pl.BlockSpec((2, BNP, H), lambda i: (0, i, 0)),
            pl.BlockSpec((2, BNP), lambda i: (0, i)),
            pl.BlockSpec((1, H), lambda i: (0, 0)),
            pl.BlockSpec((H, H), lambda i: (0, 0)),
            pl.BlockSpec((H, 1), lambda i: (0, 0)),
            pl.BlockSpec((H, 1), lambda i: (0, 0)),
        ],
        out_specs=[
            pl.BlockSpec((BNP, H), lambda i: (i, 0)),
            pl.BlockSpec((BNP, 1), lambda i: (i, 0)),
            pl.BlockSpec((BNP, 1), lambda i: (i, 0)),
            pl.BlockSpec((1, 1), lambda i: (0, 0)),
            pl.BlockSpec((1, 1), lambda i: (0, 0)),
        ],
        out_shape=[
            jax.ShapeDtypeStruct((NP, H), jnp.float32),
            jax.ShapeDtypeStruct((NP, 1), jnp.float32),
            jax.ShapeDtypeStruct((NP, 1), jnp.float32),
            jax.ShapeDtypeStruct((1, 1), jnp.float32),
            jax.ShapeDtypeStruct((1, 1), jnp.float32),
        ],
    )(acc, s, b, W, a_s, a_d)


def _tf_body(a_ref, s_ref, bb_ref, b3_ref, wr_ref, br_ref,
             wm0_ref, bm0_ref, wm1_ref, bm1_ref, wl_ref, bl_ref,
             out_ref, sum_s, cnt_s, mx_s):
    i = pl.program_id(0)

    @pl.when(i == 0)
    def _():
        sum_s[...] = jnp.zeros((G, H), jnp.float32)
        cnt_s[...] = jnp.zeros((G, 1), jnp.float32)
        mx_s[...] = jnp.full((G, H), -jnp.inf, jnp.float32)

    a = a_ref[...]
    s = s_ref[...]
    denom = (s[0] + s[1] + 1e-16)[:, None]
    h = (a[0] + a[1]) / denom + b3_ref[...]
    bb = bb_ref[...]
    oh = (bb == lax.broadcasted_iota(jnp.int32, (1, G), 1)).astype(jnp.float32)
    sum_s[...] += lax.dot_general(oh, h, (((0,), (0,)), ((), ())))
    cnt_s[...] += jnp.sum(oh, axis=0)[:, None]
    for g in range(G):
        mg = jnp.where(bb == g, h, -jnp.inf)
        mx_s[g:g + 1, :] = jnp.maximum(mx_s[g:g + 1, :],
                                       jnp.max(mg, axis=0, keepdims=True))

    @pl.when(i == NB - 1)
    def _():
        mean = sum_s[...] / jnp.maximum(cnt_s[...], 1.0)
        mxf = jnp.where(jnp.isfinite(mx_s[...]), mx_s[...], 0.0)
        z = jnp.concatenate([mean, mxf], axis=1)
        z = z @ wr_ref[...] + br_ref[...]
        z = jnp.maximum(z @ wm0_ref[...] + bm0_ref[...], 0.0)
        z = jnp.maximum(z @ wm1_ref[...] + bm1_ref[...], 0.0)
        out_ref[...] = z @ wl_ref[...] + bl_ref[...]


def _tfinal(acc, s, bb, b3, Wr, br, Wm0, bm0, Wm1, bm1, Wl, bl):
    full = lambda i: (0, 0)
    return pl.pallas_call(
        _tf_body,
        grid=(NB,),
        in_specs=[
            pl.BlockSpec((2, BNP, H), lambda i: (0, i, 0)),
            pl.BlockSpec((2, BNP), lambda i: (0, i)),
            pl.BlockSpec((BNP, 1), lambda i: (i, 0)),
            pl.BlockSpec((1, H), full),
            pl.BlockSpec((2 * H, H), full),
            pl.BlockSpec((1, H), full),
            pl.BlockSpec((H, H), full),
            pl.BlockSpec((1, H), full),
            pl.BlockSpec((H, H), full),
            pl.BlockSpec((1, H), full),
            pl.BlockSpec((H, TV * FH), full),
            pl.BlockSpec((1, TV * FH), full),
        ],
        out_specs=pl.BlockSpec((G, TV * FH), full),
        out_shape=jax.ShapeDtypeStruct((G, TV * FH), jnp.float32),
        scratch_shapes=[
            pltpu.VMEM((G, H), jnp.float32),
            pltpu.VMEM((G, 1), jnp.float32),
            pltpu.VMEM((G, H), jnp.float32),
        ],
    )(acc, s, bb, b3, Wr, br, Wm0, bm0, Wm1, bm1, Wl, bl)


def _sc_make():
    mesh = plsc.VectorSubcoreMesh(core_axis_name="c", subcore_axis_name="s",
                                  num_cores=2, num_subcores=16)

    @functools.partial(
        pl.kernel,
        out_type=[
            jax.ShapeDtypeStruct((2, NP, H), jnp.float32),
            jax.ShapeDtypeStruct((2, NP), jnp.float32),
        ],
        mesh=mesh,
        compiler_params=pltpu.CompilerParams(needs_layout_passes=False,
                                             use_tc_tiling_on_sc=False),
        scratch_types=[
            pltpu.VMEM((NP,), jnp.float32),
            pltpu.VMEM((NP,), jnp.float32),
            pltpu.VMEM((16,), jnp.float32),
            pltpu.VMEM((CPT, CE), jnp.int32),
            pltpu.VMEM((CPT, CE), jnp.int32),
            pltpu.VMEM((CE,), jnp.float32),
            pltpu.VMEM((CE, H), jnp.float32),
            pltpu.VMEM((ROWS_PT,), jnp.float32),
            pltpu.VMEM_SHARED((NP, H), jnp.float32),
            pltpu.VMEM_SHARED((NP,), jnp.float32),
            pltpu.SemaphoreType.DMA,
        ],
    )
    def sc_fn(als_hbm, ald_hbm, c_hbm, h_hbm, src_hbm, dst_hbm,
              out_hbm, s_hbm,
              als_v, ald_v, c_v, src_v, dst_v, ex_v, rows_v, z640,
              out_sh, s_sh, sem_g):
        cid = lax.axis_index("c")
        tid = lax.axis_index("s")
        wid = tid * 2 + cid
        zero16 = jnp.zeros((16,), jnp.float32)

        def zrow(r, _):
            rows_v[r, pl.ds(0, 16)] = zero16
            rows_v[r, pl.ds(16, 16)] = zero16
            return 0

        lax.fori_loop(0, CE, zrow, 0)

        def z6(k, _):
            z640[pl.ds(k * 16, 16)] = zero16
            return 0

        lax.fori_loop(0, ROWS_PT // 16, z6, 0)

        pltpu.sync_copy(als_hbm, als_v)
        pltpu.sync_copy(ald_hbm, ald_v)
        pltpu.sync_copy(c_hbm, c_v)
        pltpu.sync_copy(src_hbm.at[wid], src_v)
        pltpu.sync_copy(dst_hbm.at[wid], dst_v)

        base = tid * ROWS_PT
        pltpu.sync_copy(z640, s_sh.at[pl.ds(base, ROWS_PT)])
        for k in range(ROWS_PT // CE):
            pltpu.sync_copy(rows_v, out_sh.at[pl.ds(base + k * CE, CE)])
        plsc.subcore_barrier()

        cvec = c_v[...]

        def chunk(j, _):
            g = pltpu.async_copy(h_hbm.at[src_v.at[j]], rows_v, sem_g)

            for k in range(CE // 16):
                s16 = src_v[j, pl.ds(k * 16, 16)]
                d16 = dst_v[j, pl.ds(k * 16, 16)]
                av = plsc.load_gather(als_v, [s16])
                bv = plsc.load_gather(ald_v, [d16])
                e = av + bv
                e = jnp.maximum(e, 0.2 * e)
                ex_v[pl.ds(k * 16, 16)] = jnp.exp(e - cvec)

            g.wait()

            def srow(rr, _):
                r0 = rr * 4
                for c in range(4):
                    exr = plsc.load_gather(ex_v, [lax.broadcast(r0 + c, (16,))])
                    rows_v[r0 + c, pl.ds(0, 16)] = rows_v[r0 + c, pl.ds(0, 16)] * exr
                    rows_v[r0 + c, pl.ds(16, 16)] = rows_v[r0 + c, pl.ds(16, 16)] * exr
                return 0

            lax.fori_loop(0, CE // 4, srow, 0)

            pltpu.sync_copy(ex_v, s_sh.at[dst_v.at[j]], add=True)
            pltpu.sync_copy(rows_v, out_sh.at[dst_v.at[j]], add=True)
            return 0

        lax.fori_loop(0, CPT, chunk, 0)
        plsc.subcore_barrier()

        pltpu.sync_copy(s_sh.at[pl.ds(base, ROWS_PT)], z640)
        pltpu.sync_copy(z640, s_hbm.at[cid, pl.ds(base, ROWS_PT)])
        for k in range(ROWS_PT // CE):
            pltpu.sync_copy(out_sh.at[pl.ds(base + k * CE, CE)], rows_v)
            pltpu.sync_copy(rows_v, out_hbm.at[cid, pl.ds(base + k * CE, CE)])

    return sc_fn


_SC_CACHE = []


def _sc_fn():
    if not _SC_CACHE:
        _SC_CACHE.append(_sc_make())
    return _SC_CACHE[0]


def _sc_layer(h, als, ald, ma, mb, srcs, dsts):
    m = ma[0, 0] + mb[0, 0]
    c = jnp.where(m >= 0.0, m, 0.2 * m)
    c16 = jnp.full((16,), c, jnp.float32)
    return _sc_fn()(als.reshape(NP), ald.reshape(NP), c16, h, srcs, dsts)


def kernel(x, edge_index, batch, W1, as1, ad1, b1, W2, as2, ad2, b2,
           W3, as3, ad3, b3, Wr, br, Wm0, bm0, Wm1, bm1, Wl, bl):
    loops = jnp.arange(N, dtype=edge_index.dtype)
    src = jnp.concatenate([edge_index[0], loops,
                           jnp.full((EPAD - EP,), N, edge_index.dtype)])
    dst = jnp.concatenate([edge_index[1], loops,
                           jnp.full((EPAD - EP,), N, edge_index.dtype)])
    srcs = src.reshape(NW, CPT, CE)
    dsts = dst.reshape(NW, CPT, CE)
    x_p = jnp.concatenate([x, jnp.zeros((NP - N, D), jnp.float32)], axis=0)
    batch_p = jnp.concatenate([batch, jnp.full((NP - N,), G, batch.dtype)])
    batch_p = batch_p.reshape(NP, 1)

    h1, als1, ald1, ma1, mb1 = _t1(x_p, W1, as1.reshape(H, 1), ad1.reshape(H, 1))
    acc, s = _sc_layer(h1, als1, ald1, ma1, mb1, srcs, dsts)

    h2, als2, ald2, ma2, mb2 = _tmid(acc, s, b1.reshape(1, H), W2,
                                     as2.reshape(H, 1), ad2.reshape(H, 1))
    acc, s = _sc_layer(h2, als2, ald2, ma2, mb2, srcs, dsts)

    h3, als3, ald3, ma3, mb3 = _tmid(acc, s, b2.reshape(1, H), W3,
                                     as3.reshape(H, 1), ad3.reshape(H, 1))
    acc, s = _sc_layer(h3, als3, ald3, ma3, mb3, srcs, dsts)

    z = _tfinal(acc, s, batch_p, b3.reshape(1, H),
                Wr, br.reshape(1, H), Wm0, bm0.reshape(1, H),
                Wm1, bm1.reshape(1, H), Wl, bl.reshape(1, TV * FH))
    return z.reshape(G, FH, TV)
